# baseline (speedup 1.0000x reference)
import functools
import jax
import jax.numpy as jnp
from jax import lax
from jax.experimental import pallas as pl
from jax.experimental.pallas import tpu as pltpu
from jax.experimental.pallas import tpu_sc as plsc

_NC, _NS = 2, 16
_NW = _NC * _NS
_R = 32


def _sc_body(B, T, D, x_hbm, emb_hbm, out_hbm, ebuf, xbuf, sem):
    c = lax.axis_index("c")
    s = lax.axis_index("s")
    wid = c * _NS + s
    t_per_w = T // _NW
    n_chunks = t_per_w // _R
    W = _R * D

    def body(g, carry):
        t0 = (wid * t_per_w + g * _R) * D
        pltpu.sync_copy(emb_hbm.at[pl.ds(t0, W)], ebuf)
        for b in range(B):
            o0 = b * T * D + t0
            pltpu.sync_copy(x_hbm.at[pl.ds(o0, W)], xbuf)

            @plsc.parallel_loop(0, W // 16, unroll=8)
            def cbody(i):
                sl = pl.ds(i * 16, 16)
                plsc.addupdate(xbuf.at[sl], ebuf[sl])

            pltpu.sync_copy(xbuf, out_hbm.at[pl.ds(o0, W)])
        return carry

    lax.fori_loop(0, n_chunks, body, 0)


def kernel(x, emb):
    B, T, D = x.shape
    k = pl.kernel(
        functools.partial(_sc_body, B, T, D),
        out_type=jax.ShapeDtypeStruct((B * T * D,), x.dtype),
        mesh=plsc.VectorSubcoreMesh(
            core_axis_name="c", subcore_axis_name="s",
            num_cores=_NC, num_subcores=_NS),
        scratch_types=[
            pltpu.VMEM((_R * D,), jnp.float32),
            pltpu.VMEM((_R * D,), jnp.float32),
            pltpu.SemaphoreType.DMA,
        ],
    )
    out = k(x.reshape(-1), emb.reshape(-1))
    return out.reshape(B, T, D)
